# baseline (speedup 1.0000x reference)
import functools

import jax
import jax.numpy as jnp
from jax import lax
from jax.experimental import pallas as pl
from jax.experimental.pallas import tpu as pltpu
from jax.experimental.pallas import tpu_sc as plsc

NC = 2
NS = 16
NW = NC * NS
CH = 128
NB = 8


@functools.lru_cache(maxsize=None)
def _build(N, D):
    assert N % (NW * CH) == 0
    b_per_w = N // NW
    nch = b_per_w // CH
    ngroups = nch // NB
    assert nch % NB == 0
    mesh = plsc.VectorSubcoreMesh(core_axis_name="c", subcore_axis_name="s")

    @functools.partial(
        pl.kernel,
        out_type=jax.ShapeDtypeStruct((N, D), jnp.float32),
        mesh=mesh,
        compiler_params=pltpu.CompilerParams(use_tc_tiling_on_sc=False),
        scratch_types=[
            pltpu.VMEM((nch, CH), jnp.int32),
            pltpu.VMEM((NB, CH, D), jnp.float32),
        ] + [pltpu.SemaphoreType.DMA] * (2 * NB),
    )
    def emb(idx_hbm, table_hbm, out_hbm, idx_v, rows_v, *sems):
        gsems, wsems = sems[:NB], sems[NB:]
        wid = lax.axis_index("s") * NC + lax.axis_index("c")
        base = wid * b_per_w
        pltpu.sync_copy(idx_hbm.at[pl.ds(wid * nch, nch)], idx_v)

        for b in range(NB):
            pltpu.async_copy(table_hbm.at[idx_v.at[b]], rows_v.at[b], gsems[b])

        def group(p, carry):
            for b in range(NB):
                j = p * NB + b
                pltpu.make_async_copy(
                    table_hbm.at[idx_v.at[j]], rows_v.at[b], gsems[b]
                ).wait()
                pltpu.async_copy(
                    rows_v.at[b], out_hbm.at[pl.ds(base + j * CH, CH)], wsems[b]
                )
                bp = (b - 1) % NB
                jp = j - 1
                jn = jp + NB

                @pl.when(jp >= 0)
                def _():
                    pltpu.make_async_copy(
                        rows_v.at[bp],
                        out_hbm.at[pl.ds(base + jp * CH, CH)],
                        wsems[bp],
                    ).wait()

                @pl.when(jnp.logical_and(jp >= 0, jn < nch))
                def _():
                    pltpu.async_copy(
                        table_hbm.at[idx_v.at[jn]], rows_v.at[bp], gsems[bp]
                    )
            return carry

        lax.fori_loop(0, ngroups, group, 0)
        bl = (nch - 1) % NB
        pltpu.make_async_copy(
            rows_v.at[bl],
            out_hbm.at[pl.ds(base + (nch - 1) * CH, CH)],
            wsems[bl],
        ).wait()

    return emb


def kernel(X, table):
    N = X.size
    D = table.shape[1]
    idx2d = X.reshape(N // CH, CH).astype(jnp.int32)
    out = _build(N, D)(idx2d, table)
    return out.reshape(*X.shape, D)


# speedup vs baseline: 1.3449x; 1.3449x over previous
import functools

import jax
import jax.numpy as jnp
from jax import lax
from jax.experimental import pallas as pl
from jax.experimental.pallas import tpu as pltpu
from jax.experimental.pallas import tpu_sc as plsc

NC = 2
NS = 16
NW = NC * NS
CH = 128
NB = 8


@functools.lru_cache(maxsize=None)
def _build(N, D, NSLOT):
    assert N % (NW * CH) == 0
    b_per_w = N // NW
    nch = b_per_w // CH
    ngroups = nch // NB
    assert nch % NB == 0
    mesh = plsc.VectorSubcoreMesh(core_axis_name="c", subcore_axis_name="s")

    @functools.partial(
        pl.kernel,
        out_type=jax.ShapeDtypeStruct((NSLOT, D), jnp.float32),
        mesh=mesh,
        compiler_params=pltpu.CompilerParams(use_tc_tiling_on_sc=False),
        scratch_types=[
            pltpu.VMEM((nch, CH), jnp.int32),
            pltpu.VMEM((nch, CH), jnp.int32),
            pltpu.VMEM((NB, CH, D), jnp.float32),
        ] + [pltpu.SemaphoreType.DMA] * (2 * NB),
    )
    def emb(idx_hbm, dst_hbm, table_hbm, out_hbm, idx_v, dst_v, rows_v, *sems):
        gsems, wsems = sems[:NB], sems[NB:]
        wid = lax.axis_index("s") * NC + lax.axis_index("c")
        pltpu.sync_copy(idx_hbm.at[pl.ds(wid * nch, nch)], idx_v)
        pltpu.sync_copy(dst_hbm.at[pl.ds(wid * nch, nch)], dst_v)

        for b in range(NB):
            pltpu.async_copy(table_hbm.at[idx_v.at[b]], rows_v.at[b], gsems[b])

        def group(p, carry):
            for b in range(NB):
                j = p * NB + b
                pltpu.make_async_copy(
                    table_hbm.at[idx_v.at[j]], rows_v.at[b], gsems[b]
                ).wait()
                pltpu.async_copy(
                    rows_v.at[b], out_hbm.at[dst_v.at[j]], wsems[b]
                )
                bp = (b - 1) % NB
                jp = j - 1
                jn = jp + NB

                @pl.when(jp >= 0)
                def _():
                    pltpu.make_async_copy(
                        rows_v.at[bp], out_hbm.at[dst_v.at[jp]], wsems[bp]
                    ).wait()

                @pl.when(jnp.logical_and(jp >= 0, jn < nch))
                def _():
                    pltpu.async_copy(
                        table_hbm.at[idx_v.at[jn]], rows_v.at[bp], gsems[bp]
                    )
            return carry

        lax.fori_loop(0, ngroups, group, 0)
        bl = (nch - 1) % NB
        pltpu.make_async_copy(
            rows_v.at[bl], out_hbm.at[dst_v.at[nch - 1]], wsems[bl]
        ).wait()

    return emb


def kernel(X, table):
    B, H = X.shape
    N = B * H
    D = table.shape[1]
    assert D == 64
    HP = ((H + 7) // 8) * 8
    idx2d = X.reshape(N // CH, CH).astype(jnp.int32)
    n = jnp.arange(N, dtype=jnp.int32)
    dst2d = (2 * (HP * (n // H) + (n % H))).reshape(N // CH, CH)
    out = _build(N, D, B * HP * 2)(idx2d, dst2d, table)
    return out.reshape(B, HP, 2 * D)[:, :H, :D]
